# baseline (speedup 1.0000x reference)
import functools

import jax
import jax.numpy as jnp
from jax import lax
from jax.experimental import pallas as pl
from jax.experimental.pallas import tpu as pltpu
from jax.experimental.pallas import tpu_sc as plsc

T = 4096
C = 1024
E = 8
ED = 4096
RT = 512
NT = 24
PBUF = NT * RT
NJ = 4
NW = 32

_SQRT_HALF = 0.7071067811865476


def _router_body(x_ref, wr_ref, bias_ref, i1_ref, i2_ref, w0_ref, w1_ref,
                 r0_ref, r1_ref, cnt_ref, carry_ref):
    step = pl.program_id(0)

    @pl.when(step == 0)
    def _():
        carry_ref[...] = jnp.zeros_like(carry_ref)

    xb = x_ref[...]
    wr = wr_ref[...]
    logits = lax.dot_general(xb.astype(jnp.bfloat16), wr.astype(jnp.bfloat16),
                             (((1,), (1,)), ((), ())),
                             preferred_element_type=jnp.float32)
    logits = logits + bias_ref[...]
    rows = logits.shape[0]
    iota8 = lax.broadcasted_iota(jnp.int32, (rows, E), 1)
    m1 = jnp.max(logits, axis=1, keepdims=True)
    i1 = jnp.min(jnp.where(logits >= m1, iota8, E + 1), axis=1, keepdims=True)
    l2 = jnp.where(iota8 == i1, -jnp.inf, logits)
    m2 = jnp.max(l2, axis=1, keepdims=True)
    i2 = jnp.min(jnp.where(l2 >= m2, iota8, E + 1), axis=1, keepdims=True)
    ew = jnp.exp(m2 - m1)
    w0 = 1.0 / (1.0 + ew)
    w1 = 1.0 - w0
    oh0 = (iota8 == i1).astype(jnp.float32)
    oh1 = (iota8 == i2).astype(jnp.float32)
    oh01 = (oh0 + oh1).astype(jnp.bfloat16)
    ir = lax.broadcasted_iota(jnp.int32, (rows, rows), 0)
    ic = lax.broadcasted_iota(jnp.int32, (rows, rows), 1)
    tril = (ic < ir).astype(jnp.bfloat16)
    pref = lax.dot_general(tril, oh01, (((1,), (0,)), ((), ())),
                           preferred_element_type=jnp.float32)
    base = pref + carry_ref[...]
    rank0 = jnp.sum(base * oh0, axis=1, keepdims=True)
    rank1 = jnp.sum(base * oh1, axis=1, keepdims=True)
    carry_ref[...] = carry_ref[...] + jnp.sum(
        oh0 + oh1, axis=0, keepdims=True)
    i1_ref[...] = i1
    i2_ref[...] = i2
    w0_ref[...] = w0
    w1_ref[...] = w1
    r0_ref[...] = rank0.astype(jnp.int32)
    r1_ref[...] = rank1.astype(jnp.int32)
    cnt_ref[...] = jnp.pad(carry_ref[...], ((0, 0), (0, 128 - E)))


def _router(xf, wr, bias):
    return pl.pallas_call(
        _router_body,
        grid=(T // 512,),
        in_specs=[
            pl.BlockSpec((512, C), lambda i: (i, 0)),
            pl.BlockSpec((E, C), lambda i: (0, 0)),
            pl.BlockSpec((1, E), lambda i: (0, 0)),
        ],
        out_specs=[
            pl.BlockSpec((512, 1), lambda i: (i, 0)),
            pl.BlockSpec((512, 1), lambda i: (i, 0)),
            pl.BlockSpec((512, 1), lambda i: (i, 0)),
            pl.BlockSpec((512, 1), lambda i: (i, 0)),
            pl.BlockSpec((512, 1), lambda i: (i, 0)),
            pl.BlockSpec((512, 1), lambda i: (i, 0)),
            pl.BlockSpec((1, 128), lambda i: (0, 0)),
        ],
        out_shape=[
            jax.ShapeDtypeStruct((T, 1), jnp.int32),
            jax.ShapeDtypeStruct((T, 1), jnp.int32),
            jax.ShapeDtypeStruct((T, 1), jnp.float32),
            jax.ShapeDtypeStruct((T, 1), jnp.float32),
            jax.ShapeDtypeStruct((T, 1), jnp.int32),
            jax.ShapeDtypeStruct((T, 1), jnp.int32),
            jax.ShapeDtypeStruct((1, 128), jnp.float32),
        ],
        scratch_shapes=[pltpu.VMEM((1, E), jnp.float32)],
    )(xf, wr, bias)


def _expand_body(a_ref, s1_ref, w1_ref):
    for p in range(2):
        slabs = []
        for q in range(2):
            acc = a_ref[0, 0, p, q] * s1_ref[0, 0, 0]
            acc = acc + a_ref[0, 1, p, q] * s1_ref[0, 1, 0]
            slabs.append(acc)
        w1_ref[0, 0, p] = jnp.concatenate(slabs, axis=1).astype(jnp.bfloat16)


def _expand_w1(A1, s1r):
    return pl.pallas_call(
        _expand_body,
        grid=(E, 2),
        in_specs=[
            pl.BlockSpec((1, 2, 2, 2), lambda e, jj: (e, 0, 0, 0)),
            pl.BlockSpec((1, 2, 1, 1024, 512), lambda e, jj: (e, 0, jj, 0, 0)),
        ],
        out_specs=pl.BlockSpec((1, 1, 2, 1024, C), lambda e, jj: (e, jj, 0, 0, 0)),
        out_shape=jax.ShapeDtypeStruct((E, 2, 2, 1024, C), jnp.bfloat16),
    )(A1, s1r)


def _ffn_body(te_ref, xd_ref, w1_ref, w2_ref, b1_ref, b2_ref, wt_ref,
              yd_ref, acc_ref):
    i = pl.program_id(0)
    j = pl.program_id(1)
    active = te_ref[NT + i]

    @pl.when(active > 0)
    def _():
        xb = xd_ref[...].astype(jnp.bfloat16)
        h = lax.dot_general(xb, w1_ref[0, 0, 0],
                            (((1,), (1,)), ((), ())),
                            preferred_element_type=jnp.float32)
        h = h + b1_ref[0, 0]
        h = 0.5 * h * (1.0 + lax.erf(h * _SQRT_HALF))
        contrib = lax.dot_general(h.astype(jnp.bfloat16),
                                  w2_ref[0, 0].astype(jnp.bfloat16),
                                  (((1,), (0,)), ((), ())),
                                  preferred_element_type=jnp.float32)

        @pl.when(j == 0)
        def _():
            acc_ref[...] = contrib

        @pl.when(j > 0)
        def _():
            acc_ref[...] = acc_ref[...] + contrib

        @pl.when(j == NJ - 1)
        def _():
            yd_ref[...] = (acc_ref[...] + b2_ref[0]) * wt_ref[...]


def _ffn(te_pack, xd, w1bf, w2r, b1r, b2r, wt):
    grid_spec = pltpu.PrefetchScalarGridSpec(
        num_scalar_prefetch=1,
        grid=(NT, NJ),
        in_specs=[
            pl.BlockSpec((RT, C), lambda i, j, te: (i, 0)),
            pl.BlockSpec((1, 1, 1, 1024, C),
                         lambda i, j, te: (te[i], j // 2, j % 2, 0, 0)),
            pl.BlockSpec((1, 1, 1024, C),
                         lambda i, j, te: (te[i], (j % 2) * 2 + j // 2, 0, 0)),
            pl.BlockSpec((1, 1, 1, ED // NJ),
                         lambda i, j, te: (te[i], (j % 2) * 2 + j // 2, 0, 0)),
            pl.BlockSpec((1, 1, C), lambda i, j, te: (te[i], 0, 0)),
            pl.BlockSpec((RT, 1), lambda i, j, te: (i, 0)),
        ],
        out_specs=pl.BlockSpec((RT, C), lambda i, j, te: (i, 0)),
        scratch_shapes=[pltpu.VMEM((RT, C), jnp.float32)],
    )
    return pl.pallas_call(
        _ffn_body,
        grid_spec=grid_spec,
        out_shape=jax.ShapeDtypeStruct((PBUF, C), jnp.float32),
        compiler_params=pltpu.CompilerParams(
            dimension_semantics=("arbitrary", "arbitrary")),
    )(te_pack, xd, w1bf, w2r, b1r, b2r, wt)


def _scatter_rows(xf, pos):
    mesh = plsc.VectorSubcoreMesh(core_axis_name="c", subcore_axis_name="s")
    a_per_w = (2 * T) // NW
    chunk = 32
    nchunk = a_per_w // chunk

    @functools.partial(
        pl.kernel,
        out_type=jax.ShapeDtypeStruct((PBUF, C), jnp.float32),
        mesh=mesh,
        scratch_types=[
            pltpu.VMEM((chunk,), jnp.int32),
            pltpu.VMEM((chunk,), jnp.int32),
            pltpu.VMEM((chunk, C), jnp.float32),
            pltpu.VMEM((chunk, C), jnp.float32),
            pltpu.SemaphoreType.DMA,
            pltpu.SemaphoreType.DMA,
            pltpu.SemaphoreType.DMA,
            pltpu.SemaphoreType.DMA,
        ],
    )
    def scat_k(xf_hbm, pos_hbm, out_hbm, i0, i1, b0, b1, g0, g1, s0, s1):
        wid = lax.axis_index("s") * 2 + lax.axis_index("c")
        abase = wid * a_per_w
        tokbase = abase % T
        idxs = (i0, i1)
        bufs = (b0, b1)
        gsems = (g0, g1)
        ssems = (s0, s1)

        def start_read(c):
            pltpu.sync_copy(pos_hbm.at[pl.ds(abase + c * chunk, chunk)],
                            idxs[c % 2])
            return pltpu.async_copy(
                xf_hbm.at[pl.ds(tokbase + c * chunk, chunk)],
                bufs[c % 2], gsems[c % 2])

        wb = [None, None]
        cp = start_read(0)
        for c in range(nchunk):
            cp.wait()
            if c + 1 < nchunk:
                if wb[(c + 1) % 2] is not None:
                    wb[(c + 1) % 2].wait()
                cp = start_read(c + 1)
            wb[c % 2] = pltpu.async_copy(
                bufs[c % 2], out_hbm.at[idxs[c % 2]], ssems[c % 2])
        wb[(nchunk - 2) % 2].wait()
        wb[(nchunk - 1) % 2].wait()

    return scat_k(xf, pos)


def _combine(yd, p0, p1):
    mesh = plsc.VectorSubcoreMesh(core_axis_name="c", subcore_axis_name="s")
    rows_per_w = T // NW
    chunk = 16
    nchunk = rows_per_w // chunk

    @functools.partial(
        pl.kernel,
        out_type=jax.ShapeDtypeStruct((T, C), jnp.float32),
        mesh=mesh,
        scratch_types=[
            pltpu.VMEM((chunk,), jnp.int32),
            pltpu.VMEM((chunk,), jnp.int32),
            pltpu.VMEM((chunk, C), jnp.float32),
            pltpu.VMEM((chunk, C), jnp.float32),
            pltpu.VMEM((chunk, C), jnp.float32),
            pltpu.VMEM((chunk, C), jnp.float32),
            pltpu.SemaphoreType.DMA,
            pltpu.SemaphoreType.DMA,
            pltpu.SemaphoreType.DMA,
        ],
    )
    def combine_k(yd_hbm, p0_hbm, p1_hbm, out_hbm, i0_v, i1_v,
                  a0, b0, a1, b1, sem0, sem1, wsem):
        wid = lax.axis_index("s") * 2 + lax.axis_index("c")
        base = wid * rows_per_w
        abufs = (a0, a1)
        bbufs = (b0, b1)

        def start_gathers(c):
            off = base + c * chunk
            pltpu.sync_copy(p0_hbm.at[pl.ds(off, chunk)], i0_v)
            pltpu.sync_copy(p1_hbm.at[pl.ds(off, chunk)], i1_v)
            cpa = pltpu.async_copy(yd_hbm.at[i0_v], abufs[c % 2], sem0)
            cpb = pltpu.async_copy(yd_hbm.at[i1_v], bbufs[c % 2], sem1)
            return cpa, cpb

        wb = [None, None]
        cps = start_gathers(0)
        for c in range(nchunk):
            cps[0].wait()
            cps[1].wait()
            if c + 1 < nchunk:
                if wb[(c + 1) % 2] is not None:
                    wb[(c + 1) % 2].wait()
                cps = start_gathers(c + 1)
            av, bv = abufs[c % 2], bbufs[c % 2]
            for r in range(chunk):
                def add_vec(k, carry):
                    sl = pl.ds(k * 16, 16)
                    av[r, sl] = av[r, sl] + bv[r, sl]
                    return carry
                lax.fori_loop(0, C // 16, add_vec, 0)
            wb[c % 2] = pltpu.async_copy(
                av, out_hbm.at[pl.ds(base + c * chunk, chunk)], wsem)
        wb[(nchunk - 2) % 2].wait()
        wb[(nchunk - 1) % 2].wait()

    return combine_k(yd, p0, p1)


def _dispatch(i1, i2, rank0, rank1, w0, w1, counts_f):
    counts = counts_f[0, :E].astype(jnp.int32)
    wf = jnp.concatenate([w0, w1])
    padded = ((counts + RT - 1) // RT) * RT
    ends_p = jnp.cumsum(padded)
    gs = ends_p - padded
    pos0 = gs[i1] + rank0
    pos1 = gs[i2] + rank1
    pos = jnp.concatenate([pos0, pos1])
    wt_of = jnp.zeros((PBUF,), jnp.float32).at[pos].set(wf)
    tile_base = jnp.arange(NT, dtype=jnp.int32) * RT
    te = jnp.searchsorted(ends_p, tile_base, side="right").astype(jnp.int32)
    te = jnp.minimum(te, E - 1)
    active = (tile_base < ends_p[-1]).astype(jnp.int32)
    te_pack = jnp.concatenate([te, active])
    return pos, wt_of.reshape(PBUF, 1), te_pack


def kernel(x, A_r, S_r, b_r, domain_routing, A1, S1, b1, W2, b2, domain_id):
    Bb, N, Cc = x.shape
    xf = x.reshape(T, C)
    wr = jnp.einsum('npq,nkl->pkql', A_r, S_r).reshape(E, C)
    bias = (b_r + domain_routing[domain_id])[None, :]

    i1c, i2c, w0c, w1c, r0c, r1c, cnt = _router(xf, wr, bias)
    pos, wt_of, te_pack = _dispatch(
        i1c.reshape(T), i2c.reshape(T), r0c.reshape(T), r1c.reshape(T),
        w0c.reshape(T), w1c.reshape(T), cnt)

    s1r = S1.reshape(E, 2, 2, 1024, 512)
    w1bf = _expand_w1(A1, s1r)

    xd = _scatter_rows(xf, pos)

    w2r = W2.reshape(E, NJ, ED // NJ, C)
    b1r = b1.reshape(E, NJ, 1, ED // NJ)
    b2r = b2.reshape(E, 1, C)
    yd = _ffn(te_pack, xd, w1bf, w2r, b1r, b2r, wt_of)

    out = _combine(yd, pos[:T], pos[T:])
    return out.reshape(Bb, N, Cc)


# speedup vs baseline: 1.0117x; 1.0117x over previous
import functools

import jax
import jax.numpy as jnp
from jax import lax
from jax.experimental import pallas as pl
from jax.experimental.pallas import tpu as pltpu
from jax.experimental.pallas import tpu_sc as plsc

T = 4096
C = 1024
E = 8
ED = 4096
RT = 512
NT = 24
PBUF = NT * RT
NJ = 4
NW = 32

_SQRT_HALF = 0.7071067811865476


def _router_body(x_ref, wr_ref, bias_ref, tril_ref, i1_ref, i2_ref, w0_ref,
                 w1_ref, r0_ref, r1_ref, cnt_ref, carry_ref):
    step = pl.program_id(0)

    @pl.when(step == 0)
    def _():
        carry_ref[...] = jnp.zeros_like(carry_ref)

    xb = x_ref[...]
    wr = wr_ref[...]
    logits = lax.dot_general(xb.astype(jnp.bfloat16), wr.astype(jnp.bfloat16),
                             (((1,), (1,)), ((), ())),
                             preferred_element_type=jnp.float32)
    logits = logits + bias_ref[...]
    rows = logits.shape[0]
    iota8 = lax.broadcasted_iota(jnp.int32, (rows, E), 1)
    m1 = jnp.max(logits, axis=1, keepdims=True)
    i1 = jnp.min(jnp.where(logits >= m1, iota8, E + 1), axis=1, keepdims=True)
    l2 = jnp.where(iota8 == i1, -jnp.inf, logits)
    m2 = jnp.max(l2, axis=1, keepdims=True)
    i2 = jnp.min(jnp.where(l2 >= m2, iota8, E + 1), axis=1, keepdims=True)
    ew = jnp.exp(m2 - m1)
    w0 = 1.0 / (1.0 + ew)
    w1 = 1.0 - w0
    oh0 = (iota8 == i1).astype(jnp.float32)
    oh1 = (iota8 == i2).astype(jnp.float32)
    oh01 = (oh0 + oh1).astype(jnp.bfloat16)
    pref = lax.dot_general(tril_ref[...], oh01, (((1,), (0,)), ((), ())),
                           preferred_element_type=jnp.float32)
    base = pref + carry_ref[...]
    rank0 = jnp.sum(base * oh0, axis=1, keepdims=True)
    rank1 = jnp.sum(base * oh1, axis=1, keepdims=True)
    carry_ref[...] = carry_ref[...] + jnp.sum(
        oh0 + oh1, axis=0, keepdims=True)
    i1_ref[...] = i1
    i2_ref[...] = i2
    w0_ref[...] = w0
    w1_ref[...] = w1
    r0_ref[...] = rank0.astype(jnp.int32)
    r1_ref[...] = rank1.astype(jnp.int32)
    cnt_ref[...] = jnp.pad(carry_ref[...], ((0, 0), (0, 128 - E)))


def _router(xf, wr, bias, tril):
    return pl.pallas_call(
        _router_body,
        grid=(T // 512,),
        in_specs=[
            pl.BlockSpec((512, C), lambda i: (i, 0)),
            pl.BlockSpec((E, C), lambda i: (0, 0)),
            pl.BlockSpec((1, E), lambda i: (0, 0)),
            pl.BlockSpec((512, 512), lambda i: (0, 0)),
        ],
        out_specs=[
            pl.BlockSpec((512, 1), lambda i: (i, 0)),
            pl.BlockSpec((512, 1), lambda i: (i, 0)),
            pl.BlockSpec((512, 1), lambda i: (i, 0)),
            pl.BlockSpec((512, 1), lambda i: (i, 0)),
            pl.BlockSpec((512, 1), lambda i: (i, 0)),
            pl.BlockSpec((512, 1), lambda i: (i, 0)),
            pl.BlockSpec((1, 128), lambda i: (0, 0)),
        ],
        out_shape=[
            jax.ShapeDtypeStruct((T, 1), jnp.int32),
            jax.ShapeDtypeStruct((T, 1), jnp.int32),
            jax.ShapeDtypeStruct((T, 1), jnp.float32),
            jax.ShapeDtypeStruct((T, 1), jnp.float32),
            jax.ShapeDtypeStruct((T, 1), jnp.int32),
            jax.ShapeDtypeStruct((T, 1), jnp.int32),
            jax.ShapeDtypeStruct((1, 128), jnp.float32),
        ],
        scratch_shapes=[pltpu.VMEM((1, E), jnp.float32)],
    )(xf, wr, bias, tril)


def _expand_body(a_ref, s1_ref, w1_ref):
    for p in range(2):
        slabs = []
        for q in range(2):
            acc = a_ref[0, 0, p, q] * s1_ref[0, 0, 0]
            acc = acc + a_ref[0, 1, p, q] * s1_ref[0, 1, 0]
            slabs.append(acc)
        w1_ref[0, 0, p] = jnp.concatenate(slabs, axis=1).astype(jnp.bfloat16)


def _expand_w1(A1, s1r):
    return pl.pallas_call(
        _expand_body,
        grid=(E, 2),
        in_specs=[
            pl.BlockSpec((1, 2, 2, 2), lambda e, jj: (e, 0, 0, 0)),
            pl.BlockSpec((1, 2, 1, 1024, 512), lambda e, jj: (e, 0, jj, 0, 0)),
        ],
        out_specs=pl.BlockSpec((1, 1, 2, 1024, C), lambda e, jj: (e, jj, 0, 0, 0)),
        out_shape=jax.ShapeDtypeStruct((E, 2, 2, 1024, C), jnp.bfloat16),
    )(A1, s1r)


def _ffn_body(te_ref, xd_ref, w1_ref, w2_ref, b1_ref, b2_ref, wt_ref,
              yd_ref, acc_ref):
    i = pl.program_id(0)
    j = pl.program_id(1)
    active = te_ref[NT + i]

    @pl.when(active > 0)
    def _():
        xb = xd_ref[...].astype(jnp.bfloat16)
        h = lax.dot_general(xb, w1_ref[0, 0, 0],
                            (((1,), (1,)), ((), ())),
                            preferred_element_type=jnp.float32)
        h = h + b1_ref[0, 0]
        h = 0.5 * h * (1.0 + lax.erf(h * _SQRT_HALF))
        contrib = lax.dot_general(h.astype(jnp.bfloat16),
                                  w2_ref[0, 0].astype(jnp.bfloat16),
                                  (((1,), (0,)), ((), ())),
                                  preferred_element_type=jnp.float32)

        @pl.when(j == 0)
        def _():
            acc_ref[...] = contrib

        @pl.when(j > 0)
        def _():
            acc_ref[...] = acc_ref[...] + contrib

        @pl.when(j == NJ - 1)
        def _():
            yd_ref[...] = (acc_ref[...] + b2_ref[0]) * wt_ref[...]


def _ffn(te_pack, xd, w1bf, w2r, b1r, b2r, wt):
    grid_spec = pltpu.PrefetchScalarGridSpec(
        num_scalar_prefetch=1,
        grid=(NT, NJ),
        in_specs=[
            pl.BlockSpec((RT, C), lambda i, j, te: (te[2 * NT + i], 0)),
            pl.BlockSpec((1, 1, 1, 1024, C),
                         lambda i, j, te: (te[i], j // 2, j % 2, 0, 0)),
            pl.BlockSpec((1, 1, 1024, C),
                         lambda i, j, te: (te[i], (j % 2) * 2 + j // 2, 0, 0)),
            pl.BlockSpec((1, 1, 1, ED // NJ),
                         lambda i, j, te: (te[i], (j % 2) * 2 + j // 2, 0, 0)),
            pl.BlockSpec((1, 1, C), lambda i, j, te: (te[i], 0, 0)),
            pl.BlockSpec((RT, 1), lambda i, j, te: (i, 0)),
        ],
        out_specs=pl.BlockSpec((RT, C), lambda i, j, te: (i, 0)),
        scratch_shapes=[pltpu.VMEM((RT, C), jnp.float32)],
    )
    return pl.pallas_call(
        _ffn_body,
        grid_spec=grid_spec,
        out_shape=jax.ShapeDtypeStruct((PBUF, C), jnp.float32),
        compiler_params=pltpu.CompilerParams(
            dimension_semantics=("arbitrary", "arbitrary")),
    )(te_pack, xd, w1bf, w2r, b1r, b2r, wt)


def _scatter_rows(xf, pos):
    mesh = plsc.VectorSubcoreMesh(core_axis_name="c", subcore_axis_name="s")
    a_per_w = (2 * T) // NW
    chunk = 32
    nchunk = a_per_w // chunk

    @functools.partial(
        pl.kernel,
        out_type=jax.ShapeDtypeStruct((PBUF, C), jnp.float32),
        mesh=mesh,
        scratch_types=[
            pltpu.VMEM((chunk,), jnp.int32),
            pltpu.VMEM((chunk,), jnp.int32),
            pltpu.VMEM((chunk, C), jnp.float32),
            pltpu.VMEM((chunk, C), jnp.float32),
            pltpu.SemaphoreType.DMA,
            pltpu.SemaphoreType.DMA,
            pltpu.SemaphoreType.DMA,
            pltpu.SemaphoreType.DMA,
        ],
    )
    def scat_k(xf_hbm, pos_hbm, out_hbm, i0, i1, b0, b1, g0, g1, s0, s1):
        wid = lax.axis_index("s") * 2 + lax.axis_index("c")
        abase = wid * a_per_w
        tokbase = abase % T
        idxs = (i0, i1)
        bufs = (b0, b1)
        gsems = (g0, g1)
        ssems = (s0, s1)

        def start_read(c):
            pltpu.sync_copy(pos_hbm.at[pl.ds(abase + c * chunk, chunk)],
                            idxs[c % 2])
            return pltpu.async_copy(
                xf_hbm.at[pl.ds(tokbase + c * chunk, chunk)],
                bufs[c % 2], gsems[c % 2])

        wb = [None, None]
        cp = start_read(0)
        for c in range(nchunk):
            cp.wait()
            if c + 1 < nchunk:
                if wb[(c + 1) % 2] is not None:
                    wb[(c + 1) % 2].wait()
                cp = start_read(c + 1)
            wb[c % 2] = pltpu.async_copy(
                bufs[c % 2], out_hbm.at[idxs[c % 2]], ssems[c % 2])
        wb[(nchunk - 2) % 2].wait()
        wb[(nchunk - 1) % 2].wait()

    return scat_k(xf, pos)


def _combine(yd, p0, p1):
    mesh = plsc.VectorSubcoreMesh(core_axis_name="c", subcore_axis_name="s")
    rows_per_w = T // NW
    chunk = 16
    nchunk = rows_per_w // chunk

    @functools.partial(
        pl.kernel,
        out_type=jax.ShapeDtypeStruct((T, C), jnp.float32),
        mesh=mesh,
        scratch_types=[
            pltpu.VMEM((chunk,), jnp.int32),
            pltpu.VMEM((chunk,), jnp.int32),
            pltpu.VMEM((chunk, C), jnp.float32),
            pltpu.VMEM((chunk, C), jnp.float32),
            pltpu.VMEM((chunk, C), jnp.float32),
            pltpu.VMEM((chunk, C), jnp.float32),
            pltpu.SemaphoreType.DMA,
            pltpu.SemaphoreType.DMA,
            pltpu.SemaphoreType.DMA,
        ],
    )
    def combine_k(yd_hbm, p0_hbm, p1_hbm, out_hbm, i0_v, i1_v,
                  a0, b0, a1, b1, sem0, sem1, wsem):
        wid = lax.axis_index("s") * 2 + lax.axis_index("c")
        base = wid * rows_per_w
        abufs = (a0, a1)
        bbufs = (b0, b1)

        def start_gathers(c):
            off = base + c * chunk
            pltpu.sync_copy(p0_hbm.at[pl.ds(off, chunk)], i0_v)
            pltpu.sync_copy(p1_hbm.at[pl.ds(off, chunk)], i1_v)
            cpa = pltpu.async_copy(yd_hbm.at[i0_v], abufs[c % 2], sem0)
            cpb = pltpu.async_copy(yd_hbm.at[i1_v], bbufs[c % 2], sem1)
            return cpa, cpb

        wb = [None, None]
        cps = start_gathers(0)
        for c in range(nchunk):
            cps[0].wait()
            cps[1].wait()
            if c + 1 < nchunk:
                if wb[(c + 1) % 2] is not None:
                    wb[(c + 1) % 2].wait()
                cps = start_gathers(c + 1)
            av, bv = abufs[c % 2], bbufs[c % 2]
            for r in range(chunk):
                def add_vec(k, carry):
                    for u in range(4):
                        sl = pl.ds(k * 64 + u * 16, 16)
                        av[r, sl] = av[r, sl] + bv[r, sl]
                    return carry
                lax.fori_loop(0, C // 64, add_vec, 0)
            wb[c % 2] = pltpu.async_copy(
                av, out_hbm.at[pl.ds(base + c * chunk, chunk)], wsem)
        wb[(nchunk - 2) % 2].wait()
        wb[(nchunk - 1) % 2].wait()

    return combine_k(yd, p0, p1)


def _dispatch(i1, i2, rank0, rank1, w0, w1, counts_f):
    counts = counts_f[0, :E].astype(jnp.int32)
    wf = jnp.concatenate([w0, w1])
    padded = ((counts + RT - 1) // RT) * RT
    ends_p = jnp.cumsum(padded)
    gs = ends_p - padded
    pos0 = gs[i1] + rank0
    pos1 = gs[i2] + rank1
    pos = jnp.concatenate([pos0, pos1])
    wt_of = jnp.zeros((PBUF,), jnp.float32).at[pos].set(wf)
    tile_base = jnp.arange(NT, dtype=jnp.int32) * RT
    te = jnp.searchsorted(ends_p, tile_base, side="right").astype(jnp.int32)
    te = jnp.minimum(te, E - 1)
    active = (tile_base < ends_p[-1]).astype(jnp.int32)
    nact = ends_p[-1] // RT
    ci = jnp.minimum(jnp.arange(NT, dtype=jnp.int32), nact - 1)
    te_pack = jnp.concatenate([te, active, ci])
    return pos, wt_of.reshape(PBUF, 1), te_pack


def kernel(x, A_r, S_r, b_r, domain_routing, A1, S1, b1, W2, b2, domain_id):
    Bb, N, Cc = x.shape
    xf = x.reshape(T, C)
    wr = jnp.einsum('npq,nkl->pkql', A_r, S_r).reshape(E, C)
    bias = (b_r + domain_routing[domain_id])[None, :]

    tril = jnp.tril(jnp.ones((512, 512), jnp.bfloat16), k=-1)
    i1c, i2c, w0c, w1c, r0c, r1c, cnt = _router(xf, wr, bias, tril)
    pos, wt_of, te_pack = _dispatch(
        i1c.reshape(T), i2c.reshape(T), r0c.reshape(T), r1c.reshape(T),
        w0c.reshape(T), w1c.reshape(T), cnt)

    s1r = S1.reshape(E, 2, 2, 1024, 512)
    w1bf = _expand_w1(A1, s1r)

    xd = _scatter_rows(xf, pos)

    w2r = W2.reshape(E, NJ, ED // NJ, C)
    b1r = b1.reshape(E, NJ, 1, ED // NJ)
    b2r = b2.reshape(E, 1, C)
    yd = _ffn(te_pack, xd, w1bf, w2r, b1r, b2r, wt_of)

    out = _combine(yd, pos[:T], pos[T:])
    return out.reshape(Bb, N, Cc)


# speedup vs baseline: 1.0758x; 1.0634x over previous
import functools

import jax
import jax.numpy as jnp
from jax import lax
from jax.experimental import pallas as pl
from jax.experimental.pallas import tpu as pltpu
from jax.experimental.pallas import tpu_sc as plsc

T = 4096
C = 1024
E = 8
ED = 4096
RT = 512
NT = 24
PBUF = NT * RT
NJ = 4
NW = 32

_SQRT_HALF = 0.7071067811865476


def _router_body(x_ref, wr_ref, bias_ref, tril_ref, i1_ref, i2_ref, w0_ref,
                 w1_ref, r0_ref, r1_ref, cnt_ref, carry_ref):
    step = pl.program_id(0)

    @pl.when(step == 0)
    def _():
        carry_ref[...] = jnp.zeros_like(carry_ref)

    xb = x_ref[...]
    wr = wr_ref[...]
    logits = lax.dot_general(xb.astype(jnp.bfloat16), wr.astype(jnp.bfloat16),
                             (((1,), (1,)), ((), ())),
                             preferred_element_type=jnp.float32)
    logits = logits + bias_ref[...]
    rows = logits.shape[0]
    iota8 = lax.broadcasted_iota(jnp.int32, (rows, E), 1)
    m1 = jnp.max(logits, axis=1, keepdims=True)
    i1 = jnp.min(jnp.where(logits >= m1, iota8, E + 1), axis=1, keepdims=True)
    l2 = jnp.where(iota8 == i1, -jnp.inf, logits)
    m2 = jnp.max(l2, axis=1, keepdims=True)
    i2 = jnp.min(jnp.where(l2 >= m2, iota8, E + 1), axis=1, keepdims=True)
    ew = jnp.exp(m2 - m1)
    w0 = 1.0 / (1.0 + ew)
    w1 = 1.0 - w0
    oh0 = (iota8 == i1).astype(jnp.float32)
    oh1 = (iota8 == i2).astype(jnp.float32)
    oh01 = (oh0 + oh1).astype(jnp.bfloat16)
    pref = lax.dot_general(tril_ref[...], oh01, (((1,), (0,)), ((), ())),
                           preferred_element_type=jnp.float32)
    base = pref + carry_ref[...]
    rank0 = jnp.sum(base * oh0, axis=1, keepdims=True)
    rank1 = jnp.sum(base * oh1, axis=1, keepdims=True)
    carry_ref[...] = carry_ref[...] + jnp.sum(
        oh0 + oh1, axis=0, keepdims=True)
    i1_ref[...] = i1
    i2_ref[...] = i2
    w0_ref[...] = w0
    w1_ref[...] = w1
    r0_ref[...] = rank0.astype(jnp.int32)
    r1_ref[...] = rank1.astype(jnp.int32)
    cnt_ref[...] = jnp.pad(carry_ref[...], ((0, 0), (0, 128 - E)))


def _router(xf, wr, bias, tril):
    return pl.pallas_call(
        _router_body,
        grid=(T // 512,),
        in_specs=[
            pl.BlockSpec((512, C), lambda i: (i, 0)),
            pl.BlockSpec((E, C), lambda i: (0, 0)),
            pl.BlockSpec((1, E), lambda i: (0, 0)),
            pl.BlockSpec((512, 512), lambda i: (0, 0)),
        ],
        out_specs=[
            pl.BlockSpec((512, 1), lambda i: (i, 0)),
            pl.BlockSpec((512, 1), lambda i: (i, 0)),
            pl.BlockSpec((512, 1), lambda i: (i, 0)),
            pl.BlockSpec((512, 1), lambda i: (i, 0)),
            pl.BlockSpec((512, 1), lambda i: (i, 0)),
            pl.BlockSpec((512, 1), lambda i: (i, 0)),
            pl.BlockSpec((1, 128), lambda i: (0, 0)),
        ],
        out_shape=[
            jax.ShapeDtypeStruct((T, 1), jnp.int32),
            jax.ShapeDtypeStruct((T, 1), jnp.int32),
            jax.ShapeDtypeStruct((T, 1), jnp.float32),
            jax.ShapeDtypeStruct((T, 1), jnp.float32),
            jax.ShapeDtypeStruct((T, 1), jnp.int32),
            jax.ShapeDtypeStruct((T, 1), jnp.int32),
            jax.ShapeDtypeStruct((1, 128), jnp.float32),
        ],
        scratch_shapes=[pltpu.VMEM((1, E), jnp.float32)],
    )(xf, wr, bias, tril)


def _expand_body(a_ref, s1_ref, w1_ref):
    for p in range(2):
        slabs = []
        for q in range(2):
            acc = a_ref[0, 0, p, q] * s1_ref[0, 0, 0]
            acc = acc + a_ref[0, 1, p, q] * s1_ref[0, 1, 0]
            slabs.append(acc)
        w1_ref[0, 0, p] = jnp.concatenate(slabs, axis=1).astype(jnp.bfloat16)


def _expand_w1(A1, s1r):
    return pl.pallas_call(
        _expand_body,
        grid=(E, 2),
        in_specs=[
            pl.BlockSpec((1, 2, 2, 2), lambda e, jj: (e, 0, 0, 0)),
            pl.BlockSpec((1, 2, 1, 1024, 512), lambda e, jj: (e, 0, jj, 0, 0)),
        ],
        out_specs=pl.BlockSpec((1, 1, 2, 1024, C), lambda e, jj: (e, jj, 0, 0, 0)),
        out_shape=jax.ShapeDtypeStruct((E, 2, 2, 1024, C), jnp.bfloat16),
    )(A1, s1r)


def _ffn_body(te_ref, xd_ref, w1_ref, w2_ref, b1_ref, b2_ref, wt_ref,
              yd_ref, acc_ref):
    i = pl.program_id(0)
    j = pl.program_id(1)
    active = te_ref[NT + i]

    @pl.when(active > 0)
    def _():
        xb = xd_ref[...].astype(jnp.bfloat16)
        parts = []
        for jj in range(2):
            h = lax.dot_general(xb, w1_ref[0, jj, 0],
                                (((1,), (1,)), ((), ())),
                                preferred_element_type=jnp.float32)
            h = h + b1_ref[0, jj]
            h = 0.5 * h * (1.0 + lax.erf(h * _SQRT_HALF))
            parts.append(lax.dot_general(
                h.astype(jnp.bfloat16),
                w2_ref[0, jj].astype(jnp.bfloat16),
                (((1,), (0,)), ((), ())),
                preferred_element_type=jnp.float32))
        contrib = parts[0] + parts[1]

        @pl.when(j == 0)
        def _():
            acc_ref[...] = contrib

        @pl.when(j > 0)
        def _():
            acc_ref[...] = acc_ref[...] + contrib

        @pl.when(j == 1)
        def _():
            yd_ref[...] = (acc_ref[...] + b2_ref[0]) * wt_ref[...]


def _ffn(te_pack, xd, w1bf, w2r, b1r, b2r, wt):
    grid_spec = pltpu.PrefetchScalarGridSpec(
        num_scalar_prefetch=1,
        grid=(NT, 2),
        in_specs=[
            pl.BlockSpec((RT, C), lambda i, j, te: (te[2 * NT + i], 0)),
            pl.BlockSpec((1, 2, 1, 1024, C),
                         lambda i, j, te: (te[i], 0, j, 0, 0)),
            pl.BlockSpec((1, 2, 1024, C),
                         lambda i, j, te: (te[i], j, 0, 0)),
            pl.BlockSpec((1, 2, 1, ED // NJ),
                         lambda i, j, te: (te[i], j, 0, 0)),
            pl.BlockSpec((1, 1, C), lambda i, j, te: (te[i], 0, 0)),
            pl.BlockSpec((RT, 1), lambda i, j, te: (i, 0)),
        ],
        out_specs=pl.BlockSpec((RT, C), lambda i, j, te: (i, 0)),
        scratch_shapes=[pltpu.VMEM((RT, C), jnp.float32)],
    )
    return pl.pallas_call(
        _ffn_body,
        grid_spec=grid_spec,
        out_shape=jax.ShapeDtypeStruct((PBUF, C), jnp.float32),
        compiler_params=pltpu.CompilerParams(
            dimension_semantics=("arbitrary", "arbitrary")),
    )(te_pack, xd, w1bf, w2r, b1r, b2r, wt)


def _scatter_rows(xf, pos):
    mesh = plsc.VectorSubcoreMesh(core_axis_name="c", subcore_axis_name="s")
    a_per_w = (2 * T) // NW
    chunk = 32
    nchunk = a_per_w // chunk

    @functools.partial(
        pl.kernel,
        out_type=jax.ShapeDtypeStruct((PBUF, C), jnp.float32),
        mesh=mesh,
        scratch_types=[
            pltpu.VMEM((chunk,), jnp.int32),
            pltpu.VMEM((chunk,), jnp.int32),
            pltpu.VMEM((chunk, C), jnp.float32),
            pltpu.VMEM((chunk, C), jnp.float32),
            pltpu.SemaphoreType.DMA,
            pltpu.SemaphoreType.DMA,
            pltpu.SemaphoreType.DMA,
            pltpu.SemaphoreType.DMA,
        ],
    )
    def scat_k(xf_hbm, pos_hbm, out_hbm, i0, i1, b0, b1, g0, g1, s0, s1):
        wid = lax.axis_index("s") * 2 + lax.axis_index("c")
        abase = wid * a_per_w
        tokbase = abase % T
        idxs = (i0, i1)
        bufs = (b0, b1)
        gsems = (g0, g1)
        ssems = (s0, s1)

        def start_read(c):
            pltpu.sync_copy(pos_hbm.at[pl.ds(abase + c * chunk, chunk)],
                            idxs[c % 2])
            return pltpu.async_copy(
                xf_hbm.at[pl.ds(tokbase + c * chunk, chunk)],
                bufs[c % 2], gsems[c % 2])

        wb = [None, None]
        cp = start_read(0)
        for c in range(nchunk):
            cp.wait()
            if c + 1 < nchunk:
                if wb[(c + 1) % 2] is not None:
                    wb[(c + 1) % 2].wait()
                cp = start_read(c + 1)
            wb[c % 2] = pltpu.async_copy(
                bufs[c % 2], out_hbm.at[idxs[c % 2]], ssems[c % 2])
        wb[(nchunk - 2) % 2].wait()
        wb[(nchunk - 1) % 2].wait()

    return scat_k(xf, pos)


def _combine(yd, p0, p1):
    mesh = plsc.VectorSubcoreMesh(core_axis_name="c", subcore_axis_name="s")
    rows_per_w = T // NW
    chunk = 16
    nchunk = rows_per_w // chunk

    @functools.partial(
        pl.kernel,
        out_type=jax.ShapeDtypeStruct((T, C), jnp.float32),
        mesh=mesh,
        scratch_types=[
            pltpu.VMEM((chunk,), jnp.int32),
            pltpu.VMEM((chunk,), jnp.int32),
            pltpu.VMEM((chunk, C), jnp.float32),
            pltpu.VMEM((chunk, C), jnp.float32),
            pltpu.VMEM((chunk, C), jnp.float32),
            pltpu.VMEM((chunk, C), jnp.float32),
            pltpu.SemaphoreType.DMA,
            pltpu.SemaphoreType.DMA,
            pltpu.SemaphoreType.DMA,
        ],
    )
    def combine_k(yd_hbm, p0_hbm, p1_hbm, out_hbm, i0_v, i1_v,
                  a0, b0, a1, b1, sem0, sem1, wsem):
        wid = lax.axis_index("s") * 2 + lax.axis_index("c")
        base = wid * rows_per_w
        abufs = (a0, a1)
        bbufs = (b0, b1)

        def start_gathers(c):
            off = base + c * chunk
            pltpu.sync_copy(p0_hbm.at[pl.ds(off, chunk)], i0_v)
            pltpu.sync_copy(p1_hbm.at[pl.ds(off, chunk)], i1_v)
            cpa = pltpu.async_copy(yd_hbm.at[i0_v], abufs[c % 2], sem0)
            cpb = pltpu.async_copy(yd_hbm.at[i1_v], bbufs[c % 2], sem1)
            return cpa, cpb

        wb = [None, None]
        cps = start_gathers(0)
        for c in range(nchunk):
            cps[0].wait()
            cps[1].wait()
            if c + 1 < nchunk:
                if wb[(c + 1) % 2] is not None:
                    wb[(c + 1) % 2].wait()
                cps = start_gathers(c + 1)
            av, bv = abufs[c % 2], bbufs[c % 2]
            for r in range(chunk):
                def add_vec(k, carry):
                    for u in range(4):
                        sl = pl.ds(k * 64 + u * 16, 16)
                        av[r, sl] = av[r, sl] + bv[r, sl]
                    return carry
                lax.fori_loop(0, C // 64, add_vec, 0)
            wb[c % 2] = pltpu.async_copy(
                av, out_hbm.at[pl.ds(base + c * chunk, chunk)], wsem)
        wb[(nchunk - 2) % 2].wait()
        wb[(nchunk - 1) % 2].wait()

    return combine_k(yd, p0, p1)


def _dispatch(i1, i2, rank0, rank1, w0, w1, counts_f):
    counts = counts_f[0, :E].astype(jnp.int32)
    wf = jnp.concatenate([w0, w1])
    padded = ((counts + RT - 1) // RT) * RT
    ends_p = jnp.cumsum(padded)
    gs = ends_p - padded
    pos0 = gs[i1] + rank0
    pos1 = gs[i2] + rank1
    pos = jnp.concatenate([pos0, pos1])
    wt_of = jnp.zeros((PBUF,), jnp.float32).at[pos].set(wf)
    tile_base = jnp.arange(NT, dtype=jnp.int32) * RT
    te = jnp.searchsorted(ends_p, tile_base, side="right").astype(jnp.int32)
    te = jnp.minimum(te, E - 1)
    active = (tile_base < ends_p[-1]).astype(jnp.int32)
    nact = ends_p[-1] // RT
    ci = jnp.minimum(jnp.arange(NT, dtype=jnp.int32), nact - 1)
    te_pack = jnp.concatenate([te, active, ci])
    return pos, wt_of.reshape(PBUF, 1), te_pack


def kernel(x, A_r, S_r, b_r, domain_routing, A1, S1, b1, W2, b2, domain_id):
    Bb, N, Cc = x.shape
    xf = x.reshape(T, C)
    wr = jnp.einsum('npq,nkl->pkql', A_r, S_r).reshape(E, C)
    bias = (b_r + domain_routing[domain_id])[None, :]

    tril = jnp.tril(jnp.ones((512, 512), jnp.bfloat16), k=-1)
    i1c, i2c, w0c, w1c, r0c, r1c, cnt = _router(xf, wr, bias, tril)
    pos, wt_of, te_pack = _dispatch(
        i1c.reshape(T), i2c.reshape(T), r0c.reshape(T), r1c.reshape(T),
        w0c.reshape(T), w1c.reshape(T), cnt)

    s1r = S1.reshape(E, 2, 2, 1024, 512)
    w1bf = _expand_w1(A1, s1r)

    xd = _scatter_rows(xf, pos)

    w2r = W2.reshape(E, NJ, ED // NJ, C)
    b1r = b1.reshape(E, NJ, 1, ED // NJ)
    b2r = b2.reshape(E, 1, C)
    yd = _ffn(te_pack, xd, w1bf, w2r, b1r, b2r, wt_of)

    out = _combine(yd, pos[:T], pos[T:])
    return out.reshape(Bb, N, Cc)
